# baseline (speedup 1.0000x reference)
import functools

import jax
import jax.numpy as jnp
from jax import lax
from jax.experimental import pallas as pl
from jax.experimental.pallas import tpu as pltpu
from jax.experimental.pallas import tpu_sc as plsc


@functools.partial(jax.jit, static_argnums=(3, 4))
def _embed_concat(ids_flat, emb_table, soft_embeds, batch, seq_len):
    n_soft, d_model = soft_embeds.shape
    tokens = batch * seq_len
    out_rows_per_batch = n_soft + seq_len

    info = plsc.get_sparse_core_info()
    num_workers = info.num_cores * info.num_subcores
    num_cores = info.num_cores

    assert tokens % num_workers == 0
    b_per_w = tokens // num_workers
    assert num_workers % batch == 0
    w_per_batch = num_workers // batch
    assert seq_len % w_per_batch == 0

    chunk = 64
    while b_per_w % chunk:
        chunk //= 2
    n_chunks = b_per_w // chunk

    soft_chunk = None
    for cand in range(1, n_soft + 1):
        if n_soft % cand == 0 and batch * n_soft // cand <= num_workers:
            soft_chunk = cand
            break
    n_soft_workers = batch * n_soft // soft_chunk

    mesh = plsc.VectorSubcoreMesh(core_axis_name="c", subcore_axis_name="s")

    @functools.partial(
        pl.kernel,
        mesh=mesh,
        out_type=jax.ShapeDtypeStruct((batch * out_rows_per_batch, d_model),
                                      emb_table.dtype),
        compiler_params=pltpu.CompilerParams(use_tc_tiling_on_sc=False),
        scratch_types=[
            pltpu.VMEM((b_per_w,), jnp.int32),
            pltpu.VMEM((chunk, d_model), emb_table.dtype),
            pltpu.SemaphoreType.DMA,
        ],
    )
    def sc_kernel(ids_hbm, table_hbm, soft_hbm, out_hbm, idx_v, rows_v, sem):
        wid = lax.axis_index("s") * num_cores + lax.axis_index("c")
        tok0 = wid * b_per_w
        pltpu.sync_copy(ids_hbm.at[pl.ds(tok0, b_per_w)], idx_v)
        b = wid // w_per_batch
        s0 = (wid % w_per_batch) * b_per_w
        out0 = b * out_rows_per_batch + n_soft + s0
        for c in range(n_chunks):
            gat = pltpu.async_copy(
                table_hbm.at[idx_v.at[pl.ds(c * chunk, chunk)]], rows_v, sem)
            gat.wait()
            pltpu.sync_copy(rows_v, out_hbm.at[pl.ds(out0 + c * chunk, chunk)])

        @pl.when(wid < n_soft_workers)
        def _():
            j0 = wid * soft_chunk
            sb = j0 // n_soft
            r0 = j0 % n_soft
            pltpu.sync_copy(soft_hbm.at[pl.ds(r0, soft_chunk)],
                            rows_v.at[pl.ds(0, soft_chunk)])
            pltpu.sync_copy(rows_v.at[pl.ds(0, soft_chunk)],
                            out_hbm.at[pl.ds(sb * out_rows_per_batch + r0,
                                             soft_chunk)])

    return sc_kernel(ids_flat, emb_table, soft_embeds)


def kernel(input_ids, attention_mask, emb_table, soft_embeds):
    batch, seq_len = input_ids.shape
    n_soft, d_model = soft_embeds.shape
    out_flat = _embed_concat(input_ids.reshape(-1), emb_table, soft_embeds,
                             batch, seq_len)
    inputs_embeds = out_flat.reshape(batch, n_soft + seq_len, d_model)
    mask = jnp.concatenate(
        [jnp.ones((batch, n_soft), attention_mask.dtype), attention_mask],
        axis=-1)
    return inputs_embeds, mask


# speedup vs baseline: 1.0058x; 1.0058x over previous
import functools

import jax
import jax.numpy as jnp
from jax import lax
from jax.experimental import pallas as pl
from jax.experimental.pallas import tpu as pltpu
from jax.experimental.pallas import tpu_sc as plsc


@functools.partial(jax.jit, static_argnums=(3, 4))
def _embed_concat(ids_flat, emb_table, soft_embeds, batch, seq_len):
    n_soft, d_model = soft_embeds.shape
    tokens = batch * seq_len
    out_rows_per_batch = n_soft + seq_len

    info = plsc.get_sparse_core_info()
    num_workers = info.num_cores * info.num_subcores
    num_cores = info.num_cores

    assert tokens % num_workers == 0
    b_per_w = tokens // num_workers
    assert num_workers % batch == 0
    w_per_batch = num_workers // batch
    assert seq_len % w_per_batch == 0

    chunk = 32
    while b_per_w % chunk:
        chunk //= 2
    n_chunks = b_per_w // chunk
    nbuf = min(3, n_chunks)

    soft_chunk = None
    for cand in range(1, n_soft + 1):
        if n_soft % cand == 0 and batch * n_soft // cand <= num_workers:
            soft_chunk = cand
            break
    n_soft_workers = batch * n_soft // soft_chunk

    mesh = plsc.VectorSubcoreMesh(core_axis_name="c", subcore_axis_name="s")

    @functools.partial(
        pl.kernel,
        mesh=mesh,
        out_type=jax.ShapeDtypeStruct((batch * out_rows_per_batch, d_model),
                                      emb_table.dtype),
        compiler_params=pltpu.CompilerParams(use_tc_tiling_on_sc=False),
        scratch_types=[
            pltpu.VMEM((b_per_w,), jnp.int32),
            pltpu.VMEM((nbuf, chunk, d_model), emb_table.dtype),
            pltpu.SemaphoreType.DMA,
            pltpu.SemaphoreType.DMA,
        ],
    )
    def sc_kernel(ids_hbm, table_hbm, soft_hbm, out_hbm, idx_v, rows_v,
                  gsem, wsem):
        wid = lax.axis_index("s") * num_cores + lax.axis_index("c")
        tok0 = wid * b_per_w
        pltpu.sync_copy(ids_hbm.at[pl.ds(tok0, b_per_w)], idx_v)
        b = wid // w_per_batch
        s0 = (wid % w_per_batch) * b_per_w
        out0 = b * out_rows_per_batch + n_soft + s0

        def g_start(c):
            return pltpu.async_copy(
                table_hbm.at[idx_v.at[pl.ds(c * chunk, chunk)]],
                rows_v.at[c % nbuf], gsem)

        def w_start(c):
            return pltpu.async_copy(
                rows_v.at[c % nbuf],
                out_hbm.at[pl.ds(out0 + c * chunk, chunk)], wsem)

        wrs = [None] * n_chunks
        grs = [None] * n_chunks
        grs[0] = g_start(0)
        for c in range(n_chunks):
            grs[c].wait()
            wrs[c] = w_start(c)
            nxt = c + 1
            if nxt < n_chunks:
                if nxt >= nbuf:
                    wrs[nxt - nbuf].wait()
                grs[nxt] = g_start(nxt)
        for c in range(max(0, n_chunks - nbuf), n_chunks):
            wrs[c].wait()

        @pl.when(wid < n_soft_workers)
        def _():
            j0 = wid * soft_chunk
            sb = j0 // n_soft
            r0 = j0 % n_soft
            pltpu.sync_copy(soft_hbm.at[pl.ds(r0, soft_chunk)],
                            rows_v.at[0, pl.ds(0, soft_chunk)])
            pltpu.sync_copy(rows_v.at[0, pl.ds(0, soft_chunk)],
                            out_hbm.at[pl.ds(sb * out_rows_per_batch + r0,
                                             soft_chunk)])

    return sc_kernel(ids_flat, emb_table, soft_embeds)


def kernel(input_ids, attention_mask, emb_table, soft_embeds):
    batch, seq_len = input_ids.shape
    n_soft, d_model = soft_embeds.shape
    out_flat = _embed_concat(input_ids.reshape(-1), emb_table, soft_embeds,
                             batch, seq_len)
    inputs_embeds = out_flat.reshape(batch, n_soft + seq_len, d_model)
    mask = jnp.concatenate(
        [jnp.ones((batch, n_soft), attention_mask.dtype), attention_mask],
        axis=-1)
    return inputs_embeds, mask


# speedup vs baseline: 1.9867x; 1.9752x over previous
import functools

import jax
import jax.numpy as jnp
from jax import lax
from jax.experimental import pallas as pl
from jax.experimental.pallas import tpu as pltpu
from jax.experimental.pallas import tpu_sc as plsc


@functools.partial(jax.jit, static_argnums=(3, 4))
def _embed_concat(ids_flat, emb_table, soft_embeds, batch, seq_len):
    n_soft, d_model = soft_embeds.shape
    rows_pb = n_soft + seq_len
    lanes = 16
    nvec = d_model // lanes

    info = plsc.get_sparse_core_info()
    num_workers = info.num_cores * info.num_subcores
    num_cores = info.num_cores

    assert num_workers % batch == 0
    w_per_batch = num_workers // batch
    tpw = seq_len // w_per_batch
    assert tpw * w_per_batch == seq_len and tpw % 128 == 0

    js = (n_soft // 8) * 8
    jg = js + 8 - n_soft
    rt = (rows_pb // 8) * 8
    tail_n = rows_pb - rt
    n_main = seq_len - jg - tail_n
    assert 0 < jg < 8 and 0 < tail_n < 8 and js % 8 == 0
    assert n_soft <= 6 * 32

    chunk = 32
    full_c = tpw // chunk
    last_main = n_main - (w_per_batch - 1) * tpw
    last_full = last_main // chunk
    last_rem = last_main - last_full * chunk
    assert last_rem % 8 == 0 and last_full >= 1
    uni_c = min(full_c - 1, last_full)
    nbuf = 3

    mesh = plsc.VectorSubcoreMesh(core_axis_name="c", subcore_axis_name="s")

    @functools.partial(
        pl.kernel,
        mesh=mesh,
        out_type=jax.ShapeDtypeStruct((batch, rows_pb, d_model),
                                      emb_table.dtype),
        scratch_types=[
            pltpu.VMEM((tpw + 128,), jnp.int32),
            pltpu.VMEM((tpw,), jnp.int32),
            pltpu.VMEM((nbuf, chunk, d_model), emb_table.dtype),
            pltpu.VMEM((lanes, d_model), emb_table.dtype),
            pltpu.VMEM((8, d_model), emb_table.dtype),
            pltpu.VMEM((lanes,), jnp.int32),
            pltpu.SemaphoreType.DMA,
            pltpu.SemaphoreType.DMA,
        ],
    )
    def sc_kernel(ids_hbm, table_hbm, soft_hbm, out_hbm,
                  stage, shifted, gbuf, g2, jbuf, jidx, gsem, wsem):
        wid = lax.axis_index("s") * num_cores + lax.axis_index("c")
        b = wid // w_per_batch
        k = wid % w_per_batch
        tok0 = b * seq_len + k * tpw

        pltpu.sync_copy(ids_hbm.at[pl.ds(tok0, tpw)], stage.at[pl.ds(0, tpw)])

        @pl.when(k < w_per_batch - 1)
        def _():
            pltpu.sync_copy(ids_hbm.at[pl.ds(tok0 + tpw, 128)],
                            stage.at[pl.ds(tpw, 128)])

        @pl.when(k == 0)
        def _():
            pltpu.sync_copy(soft_hbm.at[pl.ds(js, n_soft - js)],
                            jbuf.at[pl.ds(0, n_soft - js)])
            for j in range(lanes // 16):
                jidx[pl.ds(16 * j, 16)] = stage[pl.ds(16 * j, 16)]
            pltpu.async_copy(table_hbm.at[jidx], g2, gsem).wait()
            for r in range(jg):
                for q in range(nvec):
                    jbuf[8 - jg + r, pl.ds(16 * q, 16)] = \
                        g2[r, pl.ds(16 * q, 16)]
            pltpu.sync_copy(jbuf, out_hbm.at[b, pl.ds(js, 8)])

        n_soft_chunks = -(-js // chunk)
        for m in range(n_soft_chunks):
            rows = min(chunk, js - m * chunk)

            @pl.when(k == 1 + m)
            def _(m=m, rows=rows):
                pltpu.sync_copy(soft_hbm.at[pl.ds(m * chunk, rows)],
                                gbuf.at[0, pl.ds(0, rows)])
                pltpu.sync_copy(gbuf.at[0, pl.ds(0, rows)],
                                out_hbm.at[b, pl.ds(m * chunk, rows)])

        for j in range(tpw // 16):
            shifted[pl.ds(16 * j, 16)] = stage[pl.ds(jg + 16 * j, 16)]

        row0 = js + 8 + k * tpw

        def g_start(c, n):
            return pltpu.async_copy(
                table_hbm.at[shifted.at[pl.ds(c * chunk, n)]],
                gbuf.at[c % nbuf, pl.ds(0, n)], gsem)

        def w_start(c, n):
            return pltpu.async_copy(
                gbuf.at[c % nbuf, pl.ds(0, n)],
                out_hbm.at[b, pl.ds(row0 + c * chunk, n)], wsem)

        wrs = [None] * uni_c
        grs = [None] * uni_c
        grs[0] = g_start(0, chunk)
        for c in range(uni_c):
            grs[c].wait()
            wrs[c] = w_start(c, chunk)
            nxt = c + 1
            if nxt < uni_c:
                if nxt >= nbuf:
                    wrs[nxt - nbuf].wait()
                grs[nxt] = g_start(nxt, chunk)
        for c in range(max(0, uni_c - nbuf), uni_c):
            wrs[c].wait()

        @pl.when(k < w_per_batch - 1)
        def _():
            for c in range(uni_c, full_c):
                g_start(c, chunk).wait()
                w_start(c, chunk).wait()

        @pl.when(k == w_per_batch - 1)
        def _():
            for c in range(uni_c, last_full):
                g_start(c, chunk).wait()
                w_start(c, chunk).wait()
            if last_rem:
                g_start(last_full, last_rem).wait()
                w_start(last_full, last_rem).wait()
            jidx[pl.ds(0, 16)] = stage[pl.ds(tpw - 16, 16)]
            pltpu.async_copy(table_hbm.at[jidx], g2, gsem).wait()
            for r in range(tail_n):
                for q in range(nvec):
                    jbuf[r, pl.ds(16 * q, 16)] = \
                        g2[16 - tail_n + r, pl.ds(16 * q, 16)]
            pltpu.sync_copy(jbuf.at[pl.ds(0, tail_n)],
                            out_hbm.at[b, pl.ds(rt, tail_n)])

    return sc_kernel(ids_flat, emb_table, soft_embeds)


def kernel(input_ids, attention_mask, emb_table, soft_embeds):
    batch, seq_len = input_ids.shape
    inputs_embeds = _embed_concat(input_ids.reshape(-1), emb_table,
                                  soft_embeds, batch, seq_len)
    n_soft = soft_embeds.shape[0]
    mask = jnp.concatenate(
        [jnp.ones((batch, n_soft), attention_mask.dtype), attention_mask],
        axis=-1)
    return inputs_embeds, mask


# speedup vs baseline: 4.1516x; 2.0897x over previous
import functools

import jax
import jax.numpy as jnp
from jax import lax
from jax.experimental import pallas as pl
from jax.experimental.pallas import tpu as pltpu
from jax.experimental.pallas import tpu_sc as plsc


@functools.partial(jax.jit, static_argnums=(4, 5))
def _embed_concat(ids_t, soft_idx, emb_table, soft_embeds, batch, seq_len):
    n_soft, d_model = soft_embeds.shape
    rows = n_soft + seq_len

    info = plsc.get_sparse_core_info()
    num_workers = info.num_cores * info.num_subcores
    num_cores = info.num_cores

    assert seq_len % num_workers == 0
    r_per_w = seq_len // num_workers
    chunk = 8
    while r_per_w % chunk:
        chunk //= 2
    n_chunks = r_per_w // chunk
    nbuf = min(3, n_chunks)

    s_per_w = 4
    while n_soft % s_per_w or n_soft // s_per_w > num_workers:
        s_per_w *= 2
    n_soft_workers = n_soft // s_per_w

    mesh = plsc.VectorSubcoreMesh(core_axis_name="c", subcore_axis_name="s")

    @functools.partial(
        pl.kernel,
        mesh=mesh,
        out_type=jax.ShapeDtypeStruct((rows, batch, d_model),
                                      emb_table.dtype),
        scratch_types=[
            pltpu.VMEM((r_per_w * batch,), jnp.int32),
            pltpu.VMEM((s_per_w * batch,), jnp.int32),
            pltpu.VMEM((nbuf, chunk, batch, d_model), emb_table.dtype),
            pltpu.VMEM((s_per_w, batch, d_model), emb_table.dtype),
            pltpu.SemaphoreType.DMA,
            pltpu.SemaphoreType.DMA,
        ],
    )
    def sc_kernel(ids_hbm, sidx_hbm, table_hbm, soft_hbm, out_hbm,
                  idx_v, sidx_v, vbuf, sbuf, gsem, wsem):
        wid = lax.axis_index("s") * num_cores + lax.axis_index("c")

        @pl.when(wid < n_soft_workers)
        def _():
            pltpu.sync_copy(sidx_hbm.at[pl.ds(wid * s_per_w * batch,
                                              s_per_w * batch)], sidx_v)
            pltpu.async_copy(soft_hbm.at[sidx_v],
                             sbuf.reshape(s_per_w * batch, d_model),
                             gsem).wait()
            pltpu.sync_copy(sbuf, out_hbm.at[pl.ds(wid * s_per_w, s_per_w)])

        pltpu.sync_copy(ids_hbm.at[pl.ds(wid * r_per_w * batch,
                                         r_per_w * batch)], idx_v)
        r0 = n_soft + wid * r_per_w

        def g_start(c):
            return pltpu.async_copy(
                table_hbm.at[idx_v.at[pl.ds(c * chunk * batch,
                                            chunk * batch)]],
                vbuf.at[c % nbuf].reshape(chunk * batch, d_model), gsem)

        def w_start(c):
            return pltpu.async_copy(
                vbuf.at[c % nbuf],
                out_hbm.at[pl.ds(r0 + c * chunk, chunk)], wsem)

        wrs = [None] * n_chunks
        grs = [None] * n_chunks
        grs[0] = g_start(0)
        for c in range(n_chunks):
            grs[c].wait()
            wrs[c] = w_start(c)
            nxt = c + 1
            if nxt < n_chunks:
                if nxt >= nbuf:
                    wrs[nxt - nbuf].wait()
                grs[nxt] = g_start(nxt)
        for c in range(max(0, n_chunks - nbuf), n_chunks):
            wrs[c].wait()

    return sc_kernel(ids_t, soft_idx, emb_table, soft_embeds)


def kernel(input_ids, attention_mask, emb_table, soft_embeds):
    batch, seq_len = input_ids.shape
    n_soft = soft_embeds.shape[0]
    ids_t = input_ids.T.reshape(-1)
    soft_idx = jnp.repeat(jnp.arange(n_soft, dtype=jnp.int32), batch)
    out3 = _embed_concat(ids_t, soft_idx, emb_table, soft_embeds,
                         batch, seq_len)
    inputs_embeds = jnp.swapaxes(out3, 0, 1)
    mask = jnp.concatenate(
        [jnp.ones((batch, n_soft), attention_mask.dtype), attention_mask],
        axis=-1)
    return inputs_embeds, mask


# speedup vs baseline: 4.3360x; 1.0444x over previous
import functools

import jax
import jax.numpy as jnp
from jax import lax
from jax.experimental import pallas as pl
from jax.experimental.pallas import tpu as pltpu
from jax.experimental.pallas import tpu_sc as plsc


@functools.partial(jax.jit, static_argnums=(4, 5))
def _embed_concat(ids_t, soft_idx, emb_table, soft_embeds, batch, seq_len):
    n_soft, d_model = soft_embeds.shape
    rows = n_soft + seq_len

    info = plsc.get_sparse_core_info()
    num_workers = info.num_cores * info.num_subcores
    num_cores = info.num_cores

    assert seq_len % num_workers == 0
    r_per_w = seq_len // num_workers
    chunk = 8
    while r_per_w % chunk:
        chunk //= 2
    n_chunks = r_per_w // chunk
    nbuf = min(3, n_chunks)

    s_per_w = 4
    while n_soft % s_per_w or n_soft // s_per_w > num_workers:
        s_per_w *= 2
    n_soft_workers = n_soft // s_per_w

    mesh = plsc.VectorSubcoreMesh(core_axis_name="c", subcore_axis_name="s")

    @functools.partial(
        pl.kernel,
        mesh=mesh,
        out_type=jax.ShapeDtypeStruct((rows, batch, d_model),
                                      emb_table.dtype),
        scratch_types=[
            pltpu.VMEM((r_per_w * batch,), jnp.int32),
            pltpu.VMEM((s_per_w * batch,), jnp.int32),
            pltpu.VMEM((nbuf, chunk, batch, d_model), emb_table.dtype),
            pltpu.VMEM((s_per_w, batch, d_model), emb_table.dtype),
            pltpu.SemaphoreType.DMA,
            pltpu.SemaphoreType.DMA,
            pltpu.SemaphoreType.DMA,
        ],
    )
    def sc_kernel(ids_hbm, sidx_hbm, table_hbm, soft_hbm, out_hbm,
                  idx_v, sidx_v, vbuf, sbuf, gsem, wsem, ssem):
        wid = lax.axis_index("s") * num_cores + lax.axis_index("c")

        pltpu.sync_copy(ids_hbm.at[pl.ds(wid * r_per_w * batch,
                                         r_per_w * batch)], idx_v)

        @pl.when(wid < n_soft_workers)
        def _():
            pltpu.sync_copy(sidx_hbm.at[pl.ds(wid * s_per_w * batch,
                                              s_per_w * batch)], sidx_v)
            pltpu.async_copy(soft_hbm.at[sidx_v],
                             sbuf.reshape(s_per_w * batch, d_model), ssem)

        r0 = n_soft + wid * r_per_w

        def g_start(c):
            return pltpu.async_copy(
                table_hbm.at[idx_v.at[pl.ds(c * chunk * batch,
                                            chunk * batch)]],
                vbuf.at[c % nbuf].reshape(chunk * batch, d_model), gsem)

        def w_start(c):
            return pltpu.async_copy(
                vbuf.at[c % nbuf],
                out_hbm.at[pl.ds(r0 + c * chunk, chunk)], wsem)

        wrs = [None] * n_chunks
        grs = [None] * n_chunks
        grs[0] = g_start(0)
        if n_chunks > 1:
            grs[1] = g_start(1)
        for c in range(n_chunks):
            grs[c].wait()
            wrs[c] = w_start(c)
            nxt = c + 2
            if nxt < n_chunks:
                if nxt >= nbuf:
                    wrs[nxt - nbuf].wait()
                grs[nxt] = g_start(nxt)
        for c in range(max(0, n_chunks - nbuf), n_chunks):
            wrs[c].wait()

        @pl.when(wid < n_soft_workers)
        def _():
            pltpu.make_async_copy(
                soft_hbm.at[sidx_v],
                sbuf.reshape(s_per_w * batch, d_model), ssem).wait()
            pltpu.sync_copy(sbuf, out_hbm.at[pl.ds(wid * s_per_w, s_per_w)])

    return sc_kernel(ids_t, soft_idx, emb_table, soft_embeds)


def kernel(input_ids, attention_mask, emb_table, soft_embeds):
    batch, seq_len = input_ids.shape
    n_soft = soft_embeds.shape[0]
    ids_t = input_ids.T.reshape(-1)
    soft_idx = jnp.repeat(jnp.arange(n_soft, dtype=jnp.int32), batch)
    out3 = _embed_concat(ids_t, soft_idx, emb_table, soft_embeds,
                         batch, seq_len)
    inputs_embeds = jnp.swapaxes(out3, 0, 1)
    mask = jnp.concatenate(
        [jnp.ones((batch, n_soft), attention_mask.dtype), attention_mask],
        axis=-1)
    return inputs_embeds, mask
